# Initial kernel scaffold; baseline (speedup 1.0000x reference)
import jax
import jax.numpy as jnp
from jax.experimental import pallas as pl


def kernel(h, e, edge_index, pos_enc, atom_tables, bond_tables, pos_W, pos_b, A_W, A_b, B_W, B_b, C_W, C_b, D_W, D_b, E_W, E_b, bn_h_scale, bn_h_bias, bn_e_scale, bn_e_bias, p1_W, p1_b, p2_W, p2_b):
    raise NotImplementedError("write your pallas kernel here")


# speedup vs baseline: 1.3518x; 1.3518x over previous
import functools

import jax
import jax.numpy as jnp
import numpy as np
from jax import lax
from jax.experimental import pallas as pl
from jax.experimental.pallas import tpu as pltpu
from jax.experimental.pallas import tpu_sc as plsc

N = 10000
NE = 160000
EMB = 128
HID = 512
NUM_TASKS = 128
NUM_LAYER = 3
PE_DIM = 10
AF = 9
AV = 100
BF = 3
BV = 10

H = EMB // 2

NC = 2
NS = 16
LANES = 16

EK = 40
E_PER_TILE = NE // NS
N_CHUNKS = E_PER_TILE // EK
NDP = 10240
N_PER_TILE = NDP // NS

BLK_N = 1000
BLK_E = 2000


def _node_embed_call(h, pos_sgn, atom_cat, pos_W, pos_b, wA, bA, wDB0, wDB1,
                     wE, bDB0, bDB1, bE):
  grid = N // BLK_N

  def body(h_ref, pe_ref, at_ref, pw_ref, pb_ref, wa_ref, ba_ref, w0_ref,
           w1_ref, we_ref, b0_ref, b1_ref, be_ref,
           hn_ref, ah_ref, db0_ref, db1_ref, eh_ref):
    pe = pe_ref[...]
    acc = jnp.dot(pe, pw_ref[...], preferred_element_type=jnp.float32)
    acc = acc + pb_ref[...]
    hcol = h_ref[...]
    iota = lax.broadcasted_iota(jnp.int32, (BLK_N, AV), 1)
    for f in range(AF):
      oh = (hcol[:, f:f + 1] == iota).astype(jnp.float32)
      acc = acc + jnp.dot(oh, at_ref[pl.ds(f * AV, AV), :],
                          preferred_element_type=jnp.float32)
    hn_ref[...] = acc
    ah_ref[...] = jnp.dot(acc, wa_ref[...],
                          preferred_element_type=jnp.float32) + ba_ref[...]
    db0_ref[...] = jnp.dot(acc, w0_ref[...],
                           preferred_element_type=jnp.float32) + b0_ref[...]
    db1_ref[...] = jnp.dot(acc, w1_ref[...],
                           preferred_element_type=jnp.float32) + b1_ref[...]
    eh_ref[...] = jnp.dot(acc, we_ref[...],
                          preferred_element_type=jnp.float32) + be_ref[...]

  full = lambda shape: pl.BlockSpec(shape, lambda i: (0, 0))
  blk = lambda w: pl.BlockSpec((BLK_N, w), lambda i: (i, 0))
  return pl.pallas_call(
      body,
      grid=(grid,),
      in_specs=[blk(AF), blk(PE_DIM), full((AF * AV, EMB)),
                full((PE_DIM, EMB)), full((1, EMB)),
                full((EMB, EMB)), full((1, EMB)),
                full((EMB, EMB)), full((EMB, EMB)),
                full((EMB, EMB)),
                full((1, EMB)), full((1, EMB)),
                full((1, EMB))],
      out_specs=[blk(EMB), blk(EMB), blk(EMB), blk(EMB), blk(EMB)],
      out_shape=[jax.ShapeDtypeStruct((N, EMB), jnp.float32),
                 jax.ShapeDtypeStruct((N, EMB), jnp.float32),
                 jax.ShapeDtypeStruct((N, EMB), jnp.float32),
                 jax.ShapeDtypeStruct((N, EMB), jnp.float32),
                 jax.ShapeDtypeStruct((N, EMB), jnp.float32)],
  )(h, pos_sgn, atom_cat, pos_W, pos_b, wA, bA, wDB0, wDB1, wE,
    bDB0, bDB1, bE)


def _edge_embed_call(e, bond_cat, cw, cb):
  grid = NE // BLK_E

  def body(e_ref, bt_ref, cw_ref, cb_ref, en_ref, ce_ref):
    ecol = e_ref[...]
    iota = lax.broadcasted_iota(jnp.int32, (BLK_E, BF * BV), 1)
    m = jnp.zeros((BLK_E, BF * BV), jnp.float32)
    for f in range(BF):
      m = m + (ecol[:, f:f + 1] + f * BV == iota).astype(jnp.float32)
    en = jnp.dot(m, bt_ref[...], preferred_element_type=jnp.float32)
    en_ref[...] = en
    ce_ref[...] = jnp.dot(en, cw_ref[...],
                          preferred_element_type=jnp.float32) + cb_ref[...]

  return pl.pallas_call(
      body,
      grid=(grid,),
      in_specs=[pl.BlockSpec((BLK_E, BF), lambda i: (i, 0)),
                pl.BlockSpec((BF * BV, EMB), lambda i: (0, 0)),
                pl.BlockSpec((EMB, EMB), lambda i: (0, 0)),
                pl.BlockSpec((1, EMB), lambda i: (0, 0))],
      out_specs=[pl.BlockSpec((BLK_E, EMB), lambda i: (i, 0)),
                 pl.BlockSpec((BLK_E, EMB), lambda i: (i, 0))],
      out_shape=[jax.ShapeDtypeStruct((NE, EMB), jnp.float32),
                 jax.ShapeDtypeStruct((NE, EMB), jnp.float32)],
  )(e, bond_cat, cw, cb)


def _sc_edge_body(src_ref, dst_ref, db_ref, eh_ref, ce_ref,
                  enew_ref, nd_ref, stats_ref,
                  srcv, dstv, dbbuf, ehbuf, cebuf, enewbuf, payload,
                  statsbuf, nd_sp, sem1, sem2):
  c = lax.axis_index("c")
  s = lax.axis_index("s")
  zero16 = jnp.zeros((LANES,), jnp.float32)

  def zrow(r, carry):
    for j in range(EMB // LANES):
      dbbuf[r, pl.ds(j * LANES, LANES)] = zero16
    return carry
  lax.fori_loop(0, EK, zrow, 0)
  for q in range(N_PER_TILE // EK):
    pltpu.sync_copy(dbbuf, nd_sp.at[pl.ds(s * N_PER_TILE + q * EK, EK)])
  plsc.subcore_barrier()

  ebase = s * E_PER_TILE
  coff = c * N
  lane = lax.iota(jnp.int32, LANES)

  def chunk(i, accs):
    base = ebase + i * EK
    pltpu.sync_copy(src_ref.at[pl.ds(base, EK)], srcv)
    pltpu.sync_copy(dst_ref.at[pl.ds(base, EK)], dstv)
    g2 = pltpu.async_copy(eh_ref.at[dstv], ehbuf, sem2)
    srcv[pl.ds(0, LANES)] = srcv[pl.ds(0, LANES)] + coff
    srcv[pl.ds(LANES, LANES)] = srcv[pl.ds(LANES, LANES)] + coff
    tailsl = pl.ds(EK - LANES, LANES)
    srcv[tailsl] = srcv[tailsl] + jnp.where(lane >= 2 * LANES - (EK - LANES),
                                            coff, 0)
    g1 = pltpu.async_copy(db_ref.at[srcv], dbbuf, sem1)
    pltpu.sync_copy(ce_ref.at[pl.ds(base, EK)], cebuf)
    g1.wait()
    g2.wait()

    def row(r, a):
      a = list(a)
      for j in range(H // LANES):
        jl = pl.ds(j * LANES, LANES)
        d = dbbuf[r, jl]
        b = dbbuf[r, pl.ds(H + j * LANES, LANES)]
        eh = ehbuf[r, pl.ds(c * H + j * LANES, LANES)]
        ce = cebuf[r, pl.ds(c * H + j * LANES, LANES)]
        ev = d + eh + ce
        enewbuf[r, jl] = ev
        sg = 1.0 / (1.0 + jnp.exp(-ev))
        payload[r, jl] = b * sg
        payload[r, pl.ds(H + j * LANES, LANES)] = sg
        a[j] = a[j] + ev
        a[4 + j] = a[4 + j] + ev * ev
      return tuple(a)

    accs = lax.fori_loop(0, EK, row, accs)
    pltpu.sync_copy(enewbuf, enew_ref.at[pl.ds(c * NE + base, EK)])
    pltpu.sync_copy(payload, nd_sp.at[dstv], add=True)
    return accs

  accs = lax.fori_loop(0, N_CHUNKS, chunk, (zero16,) * 8)

  for j in range(H // LANES):
    statsbuf[0, pl.ds(j * LANES, LANES)] = accs[j]
    statsbuf[1, pl.ds(j * LANES, LANES)] = accs[4 + j]
  pltpu.sync_copy(statsbuf, stats_ref.at[c * NS + s])

  plsc.subcore_barrier()
  for q in range(N_PER_TILE // EK):
    rb = s * N_PER_TILE + q * EK
    pltpu.sync_copy(nd_sp.at[pl.ds(rb, EK)], dbbuf)
    pltpu.sync_copy(dbbuf, nd_ref.at[pl.ds(c * NDP + rb, EK)])


@functools.partial(
    pl.kernel,
    out_type=[jax.ShapeDtypeStruct((NC * NE, H), jnp.float32),
              jax.ShapeDtypeStruct((NC * NDP, EMB), jnp.float32),
              jax.ShapeDtypeStruct((NC * NS, 2, H), jnp.float32)],
    mesh=plsc.VectorSubcoreMesh(core_axis_name="c", subcore_axis_name="s",
                                num_cores=NC, num_subcores=NS),
    scratch_types=[pltpu.VMEM((EK,), jnp.int32),
                   pltpu.VMEM((EK,), jnp.int32),
                   pltpu.VMEM((EK, EMB), jnp.float32),
                   pltpu.VMEM((EK, EMB), jnp.float32),
                   pltpu.VMEM((EK, EMB), jnp.float32),
                   pltpu.VMEM((EK, H), jnp.float32),
                   pltpu.VMEM((EK, EMB), jnp.float32),
                   pltpu.VMEM((2, H), jnp.float32),
                   pltpu.VMEM_SHARED((NDP, EMB), jnp.float32),
                   pltpu.SemaphoreType.DMA,
                   pltpu.SemaphoreType.DMA],
)
def _edge_pass(src_ref, dst_ref, db_ref, eh_ref, ce_ref,
               enew_ref, nd_ref, stats_ref, *scratch):
  _sc_edge_body(src_ref, dst_ref, db_ref, eh_ref, ce_ref,
                enew_ref, nd_ref, stats_ref, *scratch)


def _node_gate_call(ah, nd_lo, nd_hi):
  grid = N // BLK_N

  def body(ah_ref, lo_ref, hi_ref, hnew_ref, st_ref):
    i = pl.program_id(0)
    lo = lo_ref[...]
    hi = hi_ref[...]
    num = jnp.concatenate([lo[:, :H], hi[:, :H]], axis=1)
    den = jnp.concatenate([lo[:, H:], hi[:, H:]], axis=1)
    hnew = ah_ref[...] + num / (den + 1e-6)
    hnew_ref[...] = hnew
    st = jnp.stack([jnp.sum(hnew, axis=0), jnp.sum(hnew * hnew, axis=0)])

    @pl.when(i == 0)
    def _():
      st_ref[...] = st

    @pl.when(i != 0)
    def _():
      st_ref[...] += st

  return pl.pallas_call(
      body,
      grid=(grid,),
      in_specs=[pl.BlockSpec((BLK_N, EMB), lambda i: (i, 0)),
                pl.BlockSpec((BLK_N, EMB), lambda i: (i, 0)),
                pl.BlockSpec((BLK_N, EMB), lambda i: (i, 0))],
      out_specs=[pl.BlockSpec((BLK_N, EMB), lambda i: (i, 0)),
                 pl.BlockSpec((2, EMB), lambda i: (0, 0))],
      out_shape=[jax.ShapeDtypeStruct((N, EMB), jnp.float32),
                 jax.ShapeDtypeStruct((2, EMB), jnp.float32)],
  )(ah, nd_lo, nd_hi)


def _bn_apply(x, st_ref, scale_ref, bias_ref, count):
  mu = st_ref[0:1, :] / count
  var = st_ref[1:2, :] / count - mu * mu
  xn = (x - mu) * lax.rsqrt(var + 1e-5) * scale_ref[...] + bias_ref[...]
  return jnp.maximum(xn, 0.0)


def _node_update_call(hn, hnew, st, scale, bias, wA, bA, wDB0, wDB1,
                      wE, bDB0, bDB1, bE):
  grid = N // BLK_N

  def body(hn_ref, hx_ref, st_ref, sc_ref, bi_ref, wa_ref, ba_ref, w0_ref,
           w1_ref, we_ref, b0_ref, b1_ref, be_ref,
           out_ref, ah_ref, db0_ref, db1_ref, eh_ref):
    hbn = _bn_apply(hx_ref[...], st_ref, sc_ref, bi_ref, float(N))
    out = hn_ref[...] + hbn
    out_ref[...] = out
    ah_ref[...] = jnp.dot(out, wa_ref[...],
                          preferred_element_type=jnp.float32) + ba_ref[...]
    db0_ref[...] = jnp.dot(out, w0_ref[...],
                           preferred_element_type=jnp.float32) + b0_ref[...]
    db1_ref[...] = jnp.dot(out, w1_ref[...],
                           preferred_element_type=jnp.float32) + b1_ref[...]
    eh_ref[...] = jnp.dot(out, we_ref[...],
                          preferred_element_type=jnp.float32) + be_ref[...]

  full = lambda shape: pl.BlockSpec(shape, lambda i: (0, 0))
  blk = lambda w: pl.BlockSpec((BLK_N, w), lambda i: (i, 0))
  return pl.pallas_call(
      body,
      grid=(grid,),
      in_specs=[blk(EMB), blk(EMB), full((2, EMB)), full((1, EMB)),
                full((1, EMB)),
                full((EMB, EMB)), full((1, EMB)),
                full((EMB, EMB)), full((EMB, EMB)),
                full((EMB, EMB)),
                full((1, EMB)), full((1, EMB)),
                full((1, EMB))],
      out_specs=[blk(EMB), blk(EMB), blk(EMB), blk(EMB), blk(EMB)],
      out_shape=[jax.ShapeDtypeStruct((N, EMB), jnp.float32),
                 jax.ShapeDtypeStruct((N, EMB), jnp.float32),
                 jax.ShapeDtypeStruct((N, EMB), jnp.float32),
                 jax.ShapeDtypeStruct((N, EMB), jnp.float32),
                 jax.ShapeDtypeStruct((N, EMB), jnp.float32)],
  )(hn, hnew, st, scale, bias, wA, bA, wDB0, wDB1, wE,
    bDB0, bDB1, bE)


def _node_final_call(hn, hnew, st, scale, bias):
  grid = N // BLK_N

  def body(hn_ref, hx_ref, st_ref, sc_ref, bi_ref, pool_ref):
    i = pl.program_id(0)
    hbn = _bn_apply(hx_ref[...], st_ref, sc_ref, bi_ref, float(N))
    out = hn_ref[...] + hbn
    psum = jnp.sum(out, axis=0)
    pmax = jnp.max(out, axis=0)

    @pl.when(i == 0)
    def _():
      pool_ref[...] = jnp.stack([psum, pmax])

    @pl.when(i != 0)
    def _():
      prev = pool_ref[...]
      pool_ref[...] = jnp.stack([prev[0] + psum,
                                 jnp.maximum(prev[1], pmax)])

  blk = lambda w: pl.BlockSpec((BLK_N, w), lambda i: (i, 0))
  full = lambda shape: pl.BlockSpec(shape, lambda i: (0, 0))
  return pl.pallas_call(
      body,
      grid=(grid,),
      in_specs=[blk(EMB), blk(EMB), full((2, EMB)), full((1, EMB)),
                full((1, EMB))],
      out_specs=[full((2, EMB))],
      out_shape=[jax.ShapeDtypeStruct((2, EMB), jnp.float32)],
  )(hn, hnew, st, scale, bias)[0]


def _edge_update_call(en, enew, stats, scale, bias, cw, cb):
  grid = NE // BLK_E

  def body(en_ref, lo_ref, hi_ref, st_ref, sc_ref, bi_ref, cw_ref, cb_ref,
           out_ref, ce_ref):
    st = st_ref[...]
    ssum = jnp.sum(st[:, 0, :], axis=0)
    ssq = jnp.sum(st[:, 1, :], axis=0)
    s0 = jnp.sum(st[:NS, 0, :], axis=0)
    q0 = jnp.sum(st[:NS, 1, :], axis=0)
    s1 = jnp.sum(st[NS:, 0, :], axis=0)
    q1 = jnp.sum(st[NS:, 1, :], axis=0)
    del ssum, ssq
    mu = jnp.concatenate([s0, s1])[None, :] / float(NE)
    ex2 = jnp.concatenate([q0, q1])[None, :] / float(NE)
    var = ex2 - mu * mu
    e_new = jnp.concatenate([lo_ref[...], hi_ref[...]], axis=1)
    xn = (e_new - mu) * lax.rsqrt(var + 1e-5) * sc_ref[...] + bi_ref[...]
    out = en_ref[...] + jnp.maximum(xn, 0.0)
    out_ref[...] = out
    ce_ref[...] = jnp.dot(out, cw_ref[...],
                          preferred_element_type=jnp.float32) + cb_ref[...]

  blk = lambda w: pl.BlockSpec((BLK_E, w), lambda i: (i, 0))
  full = lambda shape: pl.BlockSpec(shape, lambda i: (0, 0))
  return pl.pallas_call(
      body,
      grid=(grid,),
      in_specs=[blk(EMB),
                pl.BlockSpec((BLK_E, H), lambda i: (i, 0)),
                pl.BlockSpec((BLK_E, H), lambda i: (NE // BLK_E + i, 0)),
                pl.BlockSpec((NC * NS, 2, H), lambda i: (0, 0, 0)),
                full((1, EMB)), full((1, EMB)),
                full((EMB, EMB)), full((1, EMB))],
      out_specs=[blk(EMB), blk(EMB)],
      out_shape=[jax.ShapeDtypeStruct((NE, EMB), jnp.float32),
                 jax.ShapeDtypeStruct((NE, EMB), jnp.float32)],
  )(en, enew, enew, stats, scale, bias, cw, cb)


def _edge_final_call(en, enew, stats, scale, bias):
  grid = NE // BLK_E

  def body(en_ref, lo_ref, hi_ref, st_ref, sc_ref, bi_ref, pool_ref):
    i = pl.program_id(0)
    st = st_ref[...]
    s0 = jnp.sum(st[:NS, 0, :], axis=0)
    q0 = jnp.sum(st[:NS, 1, :], axis=0)
    s1 = jnp.sum(st[NS:, 0, :], axis=0)
    q1 = jnp.sum(st[NS:, 1, :], axis=0)
    mu = jnp.concatenate([s0, s1])[None, :] / float(NE)
    ex2 = jnp.concatenate([q0, q1])[None, :] / float(NE)
    var = ex2 - mu * mu
    e_new = jnp.concatenate([lo_ref[...], hi_ref[...]], axis=1)
    xn = (e_new - mu) * lax.rsqrt(var + 1e-5) * sc_ref[...] + bi_ref[...]
    out = en_ref[...] + jnp.maximum(xn, 0.0)
    psum = jnp.sum(out, axis=0)
    pmax = jnp.max(out, axis=0)

    @pl.when(i == 0)
    def _():
      pool_ref[...] = jnp.stack([psum, pmax])

    @pl.when(i != 0)
    def _():
      prev = pool_ref[...]
      pool_ref[...] = jnp.stack([prev[0] + psum,
                                 jnp.maximum(prev[1], pmax)])

  blk = lambda w: pl.BlockSpec((BLK_E, w), lambda i: (i, 0))
  full = lambda shape: pl.BlockSpec(shape, lambda i: (0, 0))
  return pl.pallas_call(
      body,
      grid=(grid,),
      in_specs=[blk(EMB),
                pl.BlockSpec((BLK_E, H), lambda i: (i, 0)),
                pl.BlockSpec((BLK_E, H), lambda i: (NE // BLK_E + i, 0)),
                pl.BlockSpec((NC * NS, 2, H), lambda i: (0, 0, 0)),
                full((1, EMB)), full((1, EMB))],
      out_specs=[full((2, EMB))],
      out_shape=[jax.ShapeDtypeStruct((2, EMB), jnp.float32)],
  )(en, enew, enew, stats, scale, bias)[0]


def _head_call(pool_n, pool_e, p1_W, p1_b, p2_W, p2_b):
  def body(pn_ref, pe_ref, w1_ref, b1_ref, w2_ref, b2_ref, out_ref):
    pn = pn_ref[...]
    pe = pe_ref[...]
    parts = [pn[0:1] / float(N), pn[0:1], pn[1:2],
             pe[0:1] / float(NE), pe[0:1], pe[1:2]]
    z = b1_ref[...]
    for i, p in enumerate(parts):
      z = z + jnp.dot(p, w1_ref[pl.ds(i * EMB, EMB), :],
                      preferred_element_type=jnp.float32)
    z = jnp.maximum(z, 0.0)
    out_ref[...] = jnp.dot(z, w2_ref[...],
                           preferred_element_type=jnp.float32) + b2_ref[...]

  full = lambda shape: pl.BlockSpec(shape, lambda i: (0, 0))
  return pl.pallas_call(
      body,
      grid=(1,),
      in_specs=[full((2, EMB)), full((2, EMB)), full((6 * EMB, HID)),
                full((1, HID)), full((HID, NUM_TASKS)),
                full((1, NUM_TASKS))],
      out_specs=[full((1, NUM_TASKS))],
      out_shape=[jax.ShapeDtypeStruct((1, NUM_TASKS), jnp.float32)],
  )(pool_n, pool_e, p1_W, p1_b, p2_W, p2_b)[0]


def kernel(h, e, edge_index, pos_enc, atom_tables, bond_tables, pos_W, pos_b,
           A_W, A_b, B_W, B_b, C_W, C_b, D_W, D_b, E_W, E_b,
           bn_h_scale, bn_h_bias, bn_e_scale, bn_e_bias,
           p1_W, p1_b, p2_W, p2_b):
  f32 = jnp.float32
  h = h.astype(jnp.int32)
  e = e.astype(jnp.int32)
  src = edge_index[0].astype(jnp.int32)
  dst = edge_index[1].astype(jnp.int32)

  flip = jax.random.randint(jax.random.key(42), (1, PE_DIM), 0, 2)
  sign = jnp.where(flip == 0, -1.0, 1.0).astype(f32)
  pos_sgn = pos_enc * sign

  atom_cat = atom_tables.reshape(AF * AV, EMB).astype(f32)
  bond_cat = bond_tables.reshape(BF * BV, EMB).astype(f32)

  row = lambda v: v.reshape(1, -1).astype(f32)

  wA, bA, wDB0, wDB1, wE, bDB0, bDB1, bE = [], [], [], [], [], [], [], []
  for l in range(NUM_LAYER):
    wA.append(A_W[l])
    bA.append(row(A_b[l]))
    wDB0.append(jnp.concatenate([D_W[l][:, :H], B_W[l][:, :H]], axis=1))
    wDB1.append(jnp.concatenate([D_W[l][:, H:], B_W[l][:, H:]], axis=1))
    bDB0.append(row(jnp.concatenate([D_b[l][:H], B_b[l][:H]])))
    bDB1.append(row(jnp.concatenate([D_b[l][H:], B_b[l][H:]])))
    wE.append(E_W[l])
    bE.append(row(E_b[l]))

  hn, ah, db0, db1, eh = _node_embed_call(
      h, pos_sgn, atom_cat, pos_W.astype(f32), row(pos_b),
      wA[0], bA[0], wDB0[0], wDB1[0], wE[0],
      bDB0[0], bDB1[0], bE[0])
  en, ce = _edge_embed_call(e, bond_cat, C_W[0], row(C_b[0]))

  pool_n = pool_e = None
  for l in range(NUM_LAYER):
    db_cat = jnp.concatenate([db0, db1], axis=0)
    enew, numden, stats = _edge_pass(src, dst, db_cat, eh, ce)
    hnew, st_h = _node_gate_call(ah, numden[0:N], numden[NDP:NDP + N])
    if l < NUM_LAYER - 1:
      hn, ah, db0, db1, eh = _node_update_call(
          hn, hnew, st_h, row(bn_h_scale[l]), row(bn_h_bias[l]),
          wA[l + 1], bA[l + 1], wDB0[l + 1], wDB1[l + 1],
          wE[l + 1], bDB0[l + 1], bDB1[l + 1], bE[l + 1])
      en, ce = _edge_update_call(
          en, enew, stats, row(bn_e_scale[l]), row(bn_e_bias[l]),
          C_W[l + 1], row(C_b[l + 1]))
    else:
      pool_n = _node_final_call(hn, hnew, st_h, row(bn_h_scale[l]),
                                row(bn_h_bias[l]))
      pool_e = _edge_final_call(en, enew, stats, row(bn_e_scale[l]),
                                row(bn_e_bias[l]))

  return _head_call(pool_n, pool_e, p1_W.astype(f32), row(p1_b),
                    p2_W.astype(f32), row(p2_b))


# speedup vs baseline: 1.6473x; 1.2186x over previous
import functools

import jax
import jax.numpy as jnp
import numpy as np
from jax import lax
from jax.experimental import pallas as pl
from jax.experimental.pallas import tpu as pltpu
from jax.experimental.pallas import tpu_sc as plsc

N = 10000
NE = 160000
EMB = 128
HID = 512
NUM_TASKS = 128
NUM_LAYER = 3
PE_DIM = 10
AF = 9
AV = 100
BF = 3
BV = 10

H = EMB // 2

NC = 2
NS = 16
LANES = 16

EK = 32
E_PER_TILE = NE // NS
N_CHUNKS = (E_PER_TILE // EK // 2) * 2
TAIL = E_PER_TILE - N_CHUNKS * EK
NDP = 10240
N_PER_TILE = NDP // NS

BLK_N = 1000
BLK_E = 2000


def _node_embed_call(h, pos_sgn, atom_cat, pos_W, pos_b, wA, bA, wDB0, wDB1,
                     wE, bDB0, bDB1, bE):
  grid = N // BLK_N

  def body(h_ref, pe_ref, at_ref, pw_ref, pb_ref, wa_ref, ba_ref, w0_ref,
           w1_ref, we_ref, b0_ref, b1_ref, be_ref,
           hn_ref, ah_ref, db0_ref, db1_ref, eh_ref):
    pe = pe_ref[...]
    acc = jnp.dot(pe, pw_ref[...], preferred_element_type=jnp.float32)
    acc = acc + pb_ref[...]
    hcol = h_ref[...]
    iota = lax.broadcasted_iota(jnp.int32, (BLK_N, AV), 1)
    for f in range(AF):
      oh = (hcol[:, f:f + 1] == iota).astype(jnp.float32)
      acc = acc + jnp.dot(oh, at_ref[pl.ds(f * AV, AV), :],
                          preferred_element_type=jnp.float32)
    hn_ref[...] = acc
    ah_ref[...] = jnp.dot(acc, wa_ref[...],
                          preferred_element_type=jnp.float32) + ba_ref[...]
    db0_ref[...] = jnp.dot(acc, w0_ref[...],
                           preferred_element_type=jnp.float32) + b0_ref[...]
    db1_ref[...] = jnp.dot(acc, w1_ref[...],
                           preferred_element_type=jnp.float32) + b1_ref[...]
    eh_ref[...] = jnp.dot(acc, we_ref[...],
                          preferred_element_type=jnp.float32) + be_ref[...]

  full = lambda shape: pl.BlockSpec(shape, lambda i: (0, 0))
  blk = lambda w: pl.BlockSpec((BLK_N, w), lambda i: (i, 0))
  return pl.pallas_call(
      body,
      grid=(grid,),
      in_specs=[blk(AF), blk(PE_DIM), full((AF * AV, EMB)),
                full((PE_DIM, EMB)), full((1, EMB)),
                full((EMB, EMB)), full((1, EMB)),
                full((EMB, EMB)), full((EMB, EMB)),
                full((EMB, EMB)),
                full((1, EMB)), full((1, EMB)),
                full((1, EMB))],
      out_specs=[blk(EMB), blk(EMB), blk(EMB), blk(EMB), blk(EMB)],
      out_shape=[jax.ShapeDtypeStruct((N, EMB), jnp.float32),
                 jax.ShapeDtypeStruct((N, EMB), jnp.float32),
                 jax.ShapeDtypeStruct((N, EMB), jnp.float32),
                 jax.ShapeDtypeStruct((N, EMB), jnp.float32),
                 jax.ShapeDtypeStruct((N, EMB), jnp.float32)],
  )(h, pos_sgn, atom_cat, pos_W, pos_b, wA, bA, wDB0, wDB1, wE,
    bDB0, bDB1, bE)


def _edge_embed_call(e, bond_cat):
  grid = NE // BLK_E

  def body(e_ref, bt_ref, en_ref):
    ecol = e_ref[...]
    iota = lax.broadcasted_iota(jnp.int32, (BLK_E, BF * BV), 1)
    m = jnp.zeros((BLK_E, BF * BV), jnp.float32)
    for f in range(BF):
      m = m + (ecol[:, f:f + 1] + f * BV == iota).astype(jnp.float32)
    en_ref[...] = jnp.dot(m, bt_ref[...], preferred_element_type=jnp.float32)

  return pl.pallas_call(
      body,
      grid=(grid,),
      in_specs=[pl.BlockSpec((BLK_E, BF), lambda i: (i, 0)),
                pl.BlockSpec((BF * BV, EMB), lambda i: (0, 0))],
      out_specs=[pl.BlockSpec((BLK_E, EMB), lambda i: (i, 0))],
      out_shape=[jax.ShapeDtypeStruct((NE, EMB), jnp.float32)],
  )(e, bond_cat)[0]


def _ce_call(en, cw_stack, cb_stack):
  grid_e = NE // BLK_E

  def body(en_ref, cw_ref, cb_ref, ce_ref):
    ce_ref[...] = jnp.dot(en_ref[...], cw_ref[0],
                          preferred_element_type=jnp.float32) + cb_ref[0]

  return pl.pallas_call(
      body,
      grid=(NC, grid_e),
      in_specs=[pl.BlockSpec((BLK_E, EMB), lambda c, i: (i, 0)),
                pl.BlockSpec((1, EMB, H), lambda c, i: (c, 0, 0)),
                pl.BlockSpec((1, 1, H), lambda c, i: (c, 0, 0))],
      out_specs=[pl.BlockSpec((BLK_E, H),
                              lambda c, i: (c * (NE // BLK_E) + i, 0))],
      out_shape=[jax.ShapeDtypeStruct((NC * NE, H), jnp.float32)],
  )(en, cw_stack, cb_stack)[0]


def _sc_edge_body(src_ref, dst_ref, db_ref, eh_ref, ce_ref,
                  enew_ref, nd_ref, stats_ref,
                  srcv0, srcv1, dstv0, dstv1, dsts0, dsts1,
                  dbb0, dbb1, ehb0, ehb1, ceb0, ceb1,
                  enewb0, enewb1, payb0, payb1, dstt, statsbuf, nd_sp,
                  isem0, isem1, gsem0, gsem1, esem0, esem1, ssem0, ssem1):
  srcv = (srcv0, srcv1)
  dstv = (dstv0, dstv1)
  dsts = (dsts0, dsts1)
  dbb = (dbb0, dbb1)
  ehb = (ehb0, ehb1)
  ceb = (ceb0, ceb1)
  enewb = (enewb0, enewb1)
  payb = (payb0, payb1)
  isem = (isem0, isem1)
  gsem = (gsem0, gsem1)
  esem = (esem0, esem1)
  ssem = (ssem0, ssem1)
  c = lax.axis_index("c")
  s = lax.axis_index("s")
  zero16 = jnp.zeros((LANES,), jnp.float32)

  def zrow(r, carry):
    for j in range(EMB // LANES):
      dbb0[r, pl.ds(j * LANES, LANES)] = zero16
    return carry
  lax.fori_loop(0, EK, zrow, 0)
  for q in range(N_PER_TILE // EK):
    pltpu.sync_copy(dbb0, nd_sp.at[pl.ds(s * N_PER_TILE + q * EK, EK)])
  plsc.subcore_barrier()

  ebase = s * E_PER_TILE
  coff = c * N

  def fire_idx(i, p):
    base = ebase + i * EK
    pltpu.async_copy(src_ref.at[pl.ds(base, EK)], srcv[p], isem[p])
    pltpu.async_copy(dst_ref.at[pl.ds(base, EK)], dstv[p], isem[p])

  def wait_idx(p):
    pltpu.make_async_copy(src_ref.at[pl.ds(0, EK)], srcv[p], isem[p]).wait()
    pltpu.make_async_copy(dst_ref.at[pl.ds(0, EK)], dstv[p], isem[p]).wait()

  def fire_gathers(i, p):
    for k in range(EK // LANES):
      sl = pl.ds(k * LANES, LANES)
      srcv[p][sl] = srcv[p][sl] + coff
    pltpu.async_copy(db_ref.at[srcv[p]], dbb[p], gsem[p])
    pltpu.async_copy(eh_ref.at[dstv[p]], ehb[p], gsem[p])
    base = ebase + i * EK
    pltpu.async_copy(ce_ref.at[pl.ds(c * NE + base, EK)], ceb[p], gsem[p])

  def wait_gathers(p):
    pltpu.make_async_copy(db_ref.at[srcv[p]], dbb[p], gsem[p]).wait()
    pltpu.make_async_copy(eh_ref.at[dstv[p]], ehb[p], gsem[p]).wait()
    pltpu.make_async_copy(ce_ref.at[pl.ds(0, EK)], ceb[p], gsem[p]).wait()

  def fire_stores(i, p):
    for k in range(EK // LANES):
      sl = pl.ds(k * LANES, LANES)
      dsts[p][sl] = dstv[p][sl]
    base = ebase + i * EK
    pltpu.async_copy(enewb[p], enew_ref.at[pl.ds(c * NE + base, EK)], esem[p])
    pltpu.async_copy(payb[p], nd_sp.at[dsts[p]], ssem[p], add=True)

  def wait_stores(p):
    pltpu.make_async_copy(enewb[p], enew_ref.at[pl.ds(0, EK)], esem[p]).wait()
    pltpu.make_async_copy(payb[p], nd_sp.at[dsts[p]], ssem[p]).wait()

  def compute(p, accs, nrows=EK):
    dbbuf, ehbuf, cebuf, enewbuf, payload = \
        dbb[p], ehb[p], ceb[p], enewb[p], payb[p]

    def row(r, a):
      a = list(a)
      for j in range(H // LANES):
        jl = pl.ds(j * LANES, LANES)
        d = dbbuf[r, jl]
        b = dbbuf[r, pl.ds(H + j * LANES, LANES)]
        eh = ehbuf[r, pl.ds(c * H + j * LANES, LANES)]
        ce = cebuf[r, jl]
        ev = d + eh + ce
        enewbuf[r, jl] = ev
        sg = 1.0 / (1.0 + jnp.exp(-ev))
        payload[r, jl] = b * sg
        payload[r, pl.ds(H + j * LANES, LANES)] = sg
        a[j] = a[j] + ev
        a[4 + j] = a[4 + j] + ev * ev
      return tuple(a)

    return lax.fori_loop(0, nrows, row, accs)

  accs = (zero16,) * 8
  pltpu.sync_copy(src_ref.at[pl.ds(ebase, EK)], srcv[0])
  pltpu.sync_copy(dst_ref.at[pl.ds(ebase, EK)], dstv[0])
  fire_gathers(0, 0)
  fire_idx(1, 1)
  wait_idx(1)
  fire_gathers(1, 1)
  wait_gathers(0)
  accs = compute(0, accs)
  fire_stores(0, 0)
  fire_idx(2, 0)
  wait_idx(0)
  fire_gathers(2, 0)
  wait_gathers(1)
  accs = compute(1, accs)
  fire_stores(1, 1)
  fire_idx(3, 1)

  def steady(t, accs):
    for p in (0, 1):
      i = 2 * t + p
      o = 1 - p
      wait_idx(o)
      fire_gathers(i + 1, o)
      wait_gathers(p)
      wait_stores(p)
      accs = compute(p, accs)
      fire_stores(i, p)
      fire_idx(i + 2, p)
    return accs

  accs = lax.fori_loop(1, N_CHUNKS // 2 - 1, steady, accs)

  wait_idx(1)
  fire_gathers(N_CHUNKS - 1, 1)
  wait_gathers(0)
  wait_stores(0)
  accs = compute(0, accs)
  fire_stores(N_CHUNKS - 2, 0)
  wait_gathers(1)
  wait_stores(1)
  accs = compute(1, accs)
  fire_stores(N_CHUNKS - 1, 1)
  wait_stores(0)

  tbase = ebase + N_CHUNKS * EK
  pltpu.sync_copy(src_ref.at[pl.ds(tbase, TAIL)], srcv[0].at[pl.ds(0, TAIL)])
  pltpu.sync_copy(dst_ref.at[pl.ds(tbase, TAIL)], dstt)
  srcv[0][pl.ds(0, TAIL)] = srcv[0][pl.ds(0, TAIL)] + coff
  pltpu.async_copy(db_ref.at[srcv[0].at[pl.ds(0, TAIL)]],
                   dbb[0].at[pl.ds(0, TAIL)], gsem[0])
  pltpu.async_copy(eh_ref.at[dstt], ehb[0].at[pl.ds(0, TAIL)], gsem[0])
  pltpu.async_copy(ce_ref.at[pl.ds(c * NE + tbase, TAIL)],
                   ceb[0].at[pl.ds(0, TAIL)], gsem[0])
  pltpu.make_async_copy(db_ref.at[pl.ds(0, TAIL)],
                        dbb[0].at[pl.ds(0, TAIL)], gsem[0]).wait()
  pltpu.make_async_copy(eh_ref.at[pl.ds(0, TAIL)],
                        ehb[0].at[pl.ds(0, TAIL)], gsem[0]).wait()
  pltpu.make_async_copy(ce_ref.at[pl.ds(0, TAIL)],
                        ceb[0].at[pl.ds(0, TAIL)], gsem[0]).wait()
  accs = compute(0, accs, nrows=TAIL)
  pltpu.sync_copy(enewb[0].at[pl.ds(0, TAIL)],
                  enew_ref.at[pl.ds(c * NE + tbase, TAIL)])
  pltpu.sync_copy(payb[0].at[pl.ds(0, TAIL)], nd_sp.at[dstt], add=True)
  wait_stores(1)

  for j in range(H // LANES):
    statsbuf[0, pl.ds(j * LANES, LANES)] = accs[j]
    statsbuf[1, pl.ds(j * LANES, LANES)] = accs[4 + j]
  pltpu.sync_copy(statsbuf, stats_ref.at[c * NS + s])

  plsc.subcore_barrier()
  for q in range(N_PER_TILE // EK):
    rb = s * N_PER_TILE + q * EK
    pltpu.sync_copy(nd_sp.at[pl.ds(rb, EK)], dbb0)
    pltpu.sync_copy(dbb0, nd_ref.at[pl.ds(c * NDP + rb, EK)])


@functools.partial(
    pl.kernel,
    out_type=[jax.ShapeDtypeStruct((NC * NE, H), jnp.float32),
              jax.ShapeDtypeStruct((NC * NDP, EMB), jnp.float32),
              jax.ShapeDtypeStruct((NC * NS, 2, H), jnp.float32)],
    mesh=plsc.VectorSubcoreMesh(core_axis_name="c", subcore_axis_name="s",
                                num_cores=NC, num_subcores=NS),
    scratch_types=([pltpu.VMEM((EK,), jnp.int32)] * 6 +
                   [pltpu.VMEM((EK, EMB), jnp.float32)] * 4 +
                   [pltpu.VMEM((EK, H), jnp.float32)] * 2 +
                   [pltpu.VMEM((EK, H), jnp.float32)] * 2 +
                   [pltpu.VMEM((EK, EMB), jnp.float32)] * 2 +
                   [pltpu.VMEM((TAIL,), jnp.int32),
                    pltpu.VMEM((2, H), jnp.float32),
                    pltpu.VMEM_SHARED((NDP, EMB), jnp.float32)] +
                   [pltpu.SemaphoreType.DMA] * 8),
)
def _edge_pass(src_ref, dst_ref, db_ref, eh_ref, ce_ref,
               enew_ref, nd_ref, stats_ref, *scratch):
  _sc_edge_body(src_ref, dst_ref, db_ref, eh_ref, ce_ref,
                enew_ref, nd_ref, stats_ref, *scratch)


def _node_gate_call(ah, nd_lo, nd_hi):
  grid = N // BLK_N

  def body(ah_ref, lo_ref, hi_ref, hnew_ref, st_ref):
    i = pl.program_id(0)
    lo = lo_ref[...]
    hi = hi_ref[...]
    num = jnp.concatenate([lo[:, :H], hi[:, :H]], axis=1)
    den = jnp.concatenate([lo[:, H:], hi[:, H:]], axis=1)
    hnew = ah_ref[...] + num / (den + 1e-6)
    hnew_ref[...] = hnew
    st = jnp.stack([jnp.sum(hnew, axis=0), jnp.sum(hnew * hnew, axis=0)])

    @pl.when(i == 0)
    def _():
      st_ref[...] = st

    @pl.when(i != 0)
    def _():
      st_ref[...] += st

  return pl.pallas_call(
      body,
      grid=(grid,),
      in_specs=[pl.BlockSpec((BLK_N, EMB), lambda i: (i, 0)),
                pl.BlockSpec((BLK_N, EMB), lambda i: (i, 0)),
                pl.BlockSpec((BLK_N, EMB), lambda i: (i, 0))],
      out_specs=[pl.BlockSpec((BLK_N, EMB), lambda i: (i, 0)),
                 pl.BlockSpec((2, EMB), lambda i: (0, 0))],
      out_shape=[jax.ShapeDtypeStruct((N, EMB), jnp.float32),
                 jax.ShapeDtypeStruct((2, EMB), jnp.float32)],
  )(ah, nd_lo, nd_hi)


def _bn_apply(x, st_ref, scale_ref, bias_ref, count):
  mu = st_ref[0:1, :] / count
  var = st_ref[1:2, :] / count - mu * mu
  xn = (x - mu) * lax.rsqrt(var + 1e-5) * scale_ref[...] + bias_ref[...]
  return jnp.maximum(xn, 0.0)


def _node_update_call(hn, hnew, st, scale, bias, wA, bA, wDB0, wDB1,
                      wE, bDB0, bDB1, bE):
  grid = N // BLK_N

  def body(hn_ref, hx_ref, st_ref, sc_ref, bi_ref, wa_ref, ba_ref, w0_ref,
           w1_ref, we_ref, b0_ref, b1_ref, be_ref,
           out_ref, ah_ref, db0_ref, db1_ref, eh_ref):
    hbn = _bn_apply(hx_ref[...], st_ref, sc_ref, bi_ref, float(N))
    out = hn_ref[...] + hbn
    out_ref[...] = out
    ah_ref[...] = jnp.dot(out, wa_ref[...],
                          preferred_element_type=jnp.float32) + ba_ref[...]
    db0_ref[...] = jnp.dot(out, w0_ref[...],
                           preferred_element_type=jnp.float32) + b0_ref[...]
    db1_ref[...] = jnp.dot(out, w1_ref[...],
                           preferred_element_type=jnp.float32) + b1_ref[...]
    eh_ref[...] = jnp.dot(out, we_ref[...],
                          preferred_element_type=jnp.float32) + be_ref[...]

  full = lambda shape: pl.BlockSpec(shape, lambda i: (0, 0))
  blk = lambda w: pl.BlockSpec((BLK_N, w), lambda i: (i, 0))
  return pl.pallas_call(
      body,
      grid=(grid,),
      in_specs=[blk(EMB), blk(EMB), full((2, EMB)), full((1, EMB)),
                full((1, EMB)),
                full((EMB, EMB)), full((1, EMB)),
                full((EMB, EMB)), full((EMB, EMB)),
                full((EMB, EMB)),
                full((1, EMB)), full((1, EMB)),
                full((1, EMB))],
      out_specs=[blk(EMB), blk(EMB), blk(EMB), blk(EMB), blk(EMB)],
      out_shape=[jax.ShapeDtypeStruct((N, EMB), jnp.float32),
                 jax.ShapeDtypeStruct((N, EMB), jnp.float32),
                 jax.ShapeDtypeStruct((N, EMB), jnp.float32),
                 jax.ShapeDtypeStruct((N, EMB), jnp.float32),
                 jax.ShapeDtypeStruct((N, EMB), jnp.float32)],
  )(hn, hnew, st, scale, bias, wA, bA, wDB0, wDB1, wE,
    bDB0, bDB1, bE)


def _node_final_call(hn, hnew, st, scale, bias):
  grid = N // BLK_N

  def body(hn_ref, hx_ref, st_ref, sc_ref, bi_ref, pool_ref):
    i = pl.program_id(0)
    hbn = _bn_apply(hx_ref[...], st_ref, sc_ref, bi_ref, float(N))
    out = hn_ref[...] + hbn
    psum = jnp.sum(out, axis=0)
    pmax = jnp.max(out, axis=0)

    @pl.when(i == 0)
    def _():
      pool_ref[...] = jnp.stack([psum, pmax])

    @pl.when(i != 0)
    def _():
      prev = pool_ref[...]
      pool_ref[...] = jnp.stack([prev[0] + psum,
                                 jnp.maximum(prev[1], pmax)])

  blk = lambda w: pl.BlockSpec((BLK_N, w), lambda i: (i, 0))
  full = lambda shape: pl.BlockSpec(shape, lambda i: (0, 0))
  return pl.pallas_call(
      body,
      grid=(grid,),
      in_specs=[blk(EMB), blk(EMB), full((2, EMB)), full((1, EMB)),
                full((1, EMB))],
      out_specs=[full((2, EMB))],
      out_shape=[jax.ShapeDtypeStruct((2, EMB), jnp.float32)],
  )(hn, hnew, st, scale, bias)[0]


def _edge_update_call(en, enew, stats, scale, bias):
  grid = NE // BLK_E

  def body(en_ref, lo_ref, hi_ref, st_ref, sc_ref, bi_ref, out_ref):
    st = st_ref[...]
    ssum = jnp.sum(st[:, 0, :], axis=0)
    ssq = jnp.sum(st[:, 1, :], axis=0)
    s0 = jnp.sum(st[:NS, 0, :], axis=0)
    q0 = jnp.sum(st[:NS, 1, :], axis=0)
    s1 = jnp.sum(st[NS:, 0, :], axis=0)
    q1 = jnp.sum(st[NS:, 1, :], axis=0)
    del ssum, ssq
    mu = jnp.concatenate([s0, s1])[None, :] / float(NE)
    ex2 = jnp.concatenate([q0, q1])[None, :] / float(NE)
    var = ex2 - mu * mu
    e_new = jnp.concatenate([lo_ref[...], hi_ref[...]], axis=1)
    xn = (e_new - mu) * lax.rsqrt(var + 1e-5) * sc_ref[...] + bi_ref[...]
    out_ref[...] = en_ref[...] + jnp.maximum(xn, 0.0)

  blk = lambda w: pl.BlockSpec((BLK_E, w), lambda i: (i, 0))
  full = lambda shape: pl.BlockSpec(shape, lambda i: (0, 0))
  return pl.pallas_call(
      body,
      grid=(grid,),
      in_specs=[blk(EMB),
                pl.BlockSpec((BLK_E, H), lambda i: (i, 0)),
                pl.BlockSpec((BLK_E, H), lambda i: (NE // BLK_E + i, 0)),
                pl.BlockSpec((NC * NS, 2, H), lambda i: (0, 0, 0)),
                full((1, EMB)), full((1, EMB))],
      out_specs=[blk(EMB)],
      out_shape=[jax.ShapeDtypeStruct((NE, EMB), jnp.float32)],
  )(en, enew, enew, stats, scale, bias)[0]


def _edge_final_call(en, enew, stats, scale, bias):
  grid = NE // BLK_E

  def body(en_ref, lo_ref, hi_ref, st_ref, sc_ref, bi_ref, pool_ref):
    i = pl.program_id(0)
    st = st_ref[...]
    s0 = jnp.sum(st[:NS, 0, :], axis=0)
    q0 = jnp.sum(st[:NS, 1, :], axis=0)
    s1 = jnp.sum(st[NS:, 0, :], axis=0)
    q1 = jnp.sum(st[NS:, 1, :], axis=0)
    mu = jnp.concatenate([s0, s1])[None, :] / float(NE)
    ex2 = jnp.concatenate([q0, q1])[None, :] / float(NE)
    var = ex2 - mu * mu
    e_new = jnp.concatenate([lo_ref[...], hi_ref[...]], axis=1)
    xn = (e_new - mu) * lax.rsqrt(var + 1e-5) * sc_ref[...] + bi_ref[...]
    out = en_ref[...] + jnp.maximum(xn, 0.0)
    psum = jnp.sum(out, axis=0)
    pmax = jnp.max(out, axis=0)

    @pl.when(i == 0)
    def _():
      pool_ref[...] = jnp.stack([psum, pmax])

    @pl.when(i != 0)
    def _():
      prev = pool_ref[...]
      pool_ref[...] = jnp.stack([prev[0] + psum,
                                 jnp.maximum(prev[1], pmax)])

  blk = lambda w: pl.BlockSpec((BLK_E, w), lambda i: (i, 0))
  full = lambda shape: pl.BlockSpec(shape, lambda i: (0, 0))
  return pl.pallas_call(
      body,
      grid=(grid,),
      in_specs=[blk(EMB),
                pl.BlockSpec((BLK_E, H), lambda i: (i, 0)),
                pl.BlockSpec((BLK_E, H), lambda i: (NE // BLK_E + i, 0)),
                pl.BlockSpec((NC * NS, 2, H), lambda i: (0, 0, 0)),
                full((1, EMB)), full((1, EMB))],
      out_specs=[full((2, EMB))],
      out_shape=[jax.ShapeDtypeStruct((2, EMB), jnp.float32)],
  )(en, enew, enew, stats, scale, bias)[0]


def _head_call(pool_n, pool_e, p1_W, p1_b, p2_W, p2_b):
  def body(pn_ref, pe_ref, w1_ref, b1_ref, w2_ref, b2_ref, out_ref):
    pn = pn_ref[...]
    pe = pe_ref[...]
    parts = [pn[0:1] / float(N), pn[0:1], pn[1:2],
             pe[0:1] / float(NE), pe[0:1], pe[1:2]]
    z = b1_ref[...]
    for i, p in enumerate(parts):
      z = z + jnp.dot(p, w1_ref[pl.ds(i * EMB, EMB), :],
                      preferred_element_type=jnp.float32)
    z = jnp.maximum(z, 0.0)
    out_ref[...] = jnp.dot(z, w2_ref[...],
                           preferred_element_type=jnp.float32) + b2_ref[...]

  full = lambda shape: pl.BlockSpec(shape, lambda i: (0, 0))
  return pl.pallas_call(
      body,
      grid=(1,),
      in_specs=[full((2, EMB)), full((2, EMB)), full((6 * EMB, HID)),
                full((1, HID)), full((HID, NUM_TASKS)),
                full((1, NUM_TASKS))],
      out_specs=[full((1, NUM_TASKS))],
      out_shape=[jax.ShapeDtypeStruct((1, NUM_TASKS), jnp.float32)],
  )(pool_n, pool_e, p1_W, p1_b, p2_W, p2_b)[0]


def kernel(h, e, edge_index, pos_enc, atom_tables, bond_tables, pos_W, pos_b,
           A_W, A_b, B_W, B_b, C_W, C_b, D_W, D_b, E_W, E_b,
           bn_h_scale, bn_h_bias, bn_e_scale, bn_e_bias,
           p1_W, p1_b, p2_W, p2_b):
  f32 = jnp.float32
  h = h.astype(jnp.int32)
  e = e.astype(jnp.int32)
  src = edge_index[0].astype(jnp.int32)
  dst = edge_index[1].astype(jnp.int32)

  flip = jax.random.randint(jax.random.key(42), (1, PE_DIM), 0, 2)
  sign = jnp.where(flip == 0, -1.0, 1.0).astype(f32)
  pos_sgn = pos_enc * sign

  atom_cat = atom_tables.reshape(AF * AV, EMB).astype(f32)
  bond_cat = bond_tables.reshape(BF * BV, EMB).astype(f32)

  row = lambda v: v.reshape(1, -1).astype(f32)

  wA, bA, wDB0, wDB1, wE, bDB0, bDB1, bE = [], [], [], [], [], [], [], []
  for l in range(NUM_LAYER):
    wA.append(A_W[l])
    bA.append(row(A_b[l]))
    wDB0.append(jnp.concatenate([D_W[l][:, :H], B_W[l][:, :H]], axis=1))
    wDB1.append(jnp.concatenate([D_W[l][:, H:], B_W[l][:, H:]], axis=1))
    bDB0.append(row(jnp.concatenate([D_b[l][:H], B_b[l][:H]])))
    bDB1.append(row(jnp.concatenate([D_b[l][H:], B_b[l][H:]])))
    wE.append(E_W[l])
    bE.append(row(E_b[l]))
  cwS = [jnp.stack([C_W[l][:, :H], C_W[l][:, H:]]) for l in range(NUM_LAYER)]
  cbS = [jnp.stack([C_b[l][:H].reshape(1, H), C_b[l][H:].reshape(1, H)])
         for l in range(NUM_LAYER)]

  hn, ah, db0, db1, eh = _node_embed_call(
      h, pos_sgn, atom_cat, pos_W.astype(f32), row(pos_b),
      wA[0], bA[0], wDB0[0], wDB1[0], wE[0],
      bDB0[0], bDB1[0], bE[0])
  en = _edge_embed_call(e, bond_cat)

  pool_n = pool_e = None
  for l in range(NUM_LAYER):
    ce = _ce_call(en, cwS[l], cbS[l])
    db_cat = jnp.concatenate([db0, db1], axis=0)
    enew, numden, stats = _edge_pass(src, dst, db_cat, eh, ce)
    hnew, st_h = _node_gate_call(ah, numden[0:N], numden[NDP:NDP + N])
    if l < NUM_LAYER - 1:
      hn, ah, db0, db1, eh = _node_update_call(
          hn, hnew, st_h, row(bn_h_scale[l]), row(bn_h_bias[l]),
          wA[l + 1], bA[l + 1], wDB0[l + 1], wDB1[l + 1],
          wE[l + 1], bDB0[l + 1], bDB1[l + 1], bE[l + 1])
      en = _edge_update_call(
          en, enew, stats, row(bn_e_scale[l]), row(bn_e_bias[l]))
    else:
      pool_n = _node_final_call(hn, hnew, st_h, row(bn_h_scale[l]),
                                row(bn_h_bias[l]))
      pool_e = _edge_final_call(en, enew, stats, row(bn_e_scale[l]),
                                row(bn_e_bias[l]))

  return _head_call(pool_n, pool_e, p1_W.astype(f32), row(p1_b),
                    p2_W.astype(f32), row(p2_b))
